# baseline (speedup 1.0000x reference)
import functools

import jax
import jax.numpy as jnp
from jax.experimental import pallas as pl

N = 10000
E = 160000
IN = 128
HID = 64
HEADS = 8
G = 128


def _mm_kernel(x_ref, w_ref, o_ref):
    o_ref[...] = jnp.dot(x_ref[...], w_ref[...],
                         preferred_element_type=jnp.float32)


def _matmul(x, w):
    m, k = x.shape
    k2, n = w.shape
    bm = 512
    grid = (pl.cdiv(m, bm),)
    return pl.pallas_call(
        _mm_kernel,
        grid=grid,
        in_specs=[
            pl.BlockSpec((bm, k), lambda i: (i, 0)),
            pl.BlockSpec((k, n), lambda i: (0, 0)),
        ],
        out_specs=pl.BlockSpec((bm, n), lambda i: (i, 0)),
        out_shape=jax.ShapeDtypeStruct((m, n), jnp.float32),
    )(x, w)


def _gat_conv(x, edge_index, W, att_src, att_dst, b, heads, out_ch):
    n = x.shape[0]
    loop = jnp.arange(n, dtype=edge_index.dtype)
    src = jnp.concatenate([edge_index[0], loop])
    dst = jnp.concatenate([edge_index[1], loop])
    npad = ((n + 511) // 512) * 512
    xp = jnp.pad(x, ((0, npad - n), (0, 0)))
    h = _matmul(xp, W)[:n].reshape(n, heads, out_ch)
    a_src = (h * att_src[None, :, :]).sum(-1)
    a_dst = (h * att_dst[None, :, :]).sum(-1)
    alpha = a_src[src] + a_dst[dst]
    alpha = jax.nn.leaky_relu(alpha, negative_slope=0.2)
    amax = jax.ops.segment_max(alpha, dst, num_segments=n)
    amax = jnp.where(jnp.isfinite(amax), amax, 0.0)
    ex = jnp.exp(alpha - amax[dst])
    denom = jax.ops.segment_sum(ex, dst, num_segments=n)
    att = ex / (denom[dst] + 1e-16)
    msg = h[src] * att[:, :, None]
    out = jax.ops.segment_sum(msg, dst, num_segments=n)
    return out.reshape(n, heads * out_ch) + b[None, :]


def kernel(x, edge_index, edge_attr, batch, W1, att_src1, att_dst1, b1,
           W2, att_src2, att_dst2, b2, lin1_w, lin1_b, lin2_w, lin2_b):
    h = jax.nn.elu(_gat_conv(x, edge_index, W1, att_src1, att_dst1, b1,
                             HEADS, HID))
    h = jax.nn.elu(_gat_conv(h, edge_index, W2, att_src2, att_dst2, b2,
                             1, HID))
    s = jax.ops.segment_sum(h, batch, num_segments=G)
    cnt = jax.ops.segment_sum(jnp.ones((h.shape[0],), dtype=h.dtype),
                              batch, num_segments=G)
    pooled = s / jnp.clip(cnt, 1.0, None)[:, None]
    z = jax.nn.relu(pooled @ lin1_w + lin1_b[None, :])
    out = (z @ lin2_w + lin2_b[None, :]).squeeze(1)
    return out


# speedup vs baseline: 18.7255x; 18.7255x over previous
import functools

import jax
import jax.numpy as jnp
from jax import lax
from jax.experimental import pallas as pl
from jax.experimental.pallas import tpu as pltpu
from jax.experimental.pallas import tpu_sc as plsc

N = 10000
E = 160000
IN = 128
HID = 64
HEADS = 8
G = 128

NC = 2
NS = 16
NW = NC * NS
CH = 128
NCHUNK = E // CH
NP = 10240
ROWS_W = NP // NS

_mesh = plsc.VectorSubcoreMesh(core_axis_name="c", subcore_axis_name="s")


def _mm_kernel(x_ref, w_ref, o_ref):
    o_ref[...] = jnp.dot(x_ref[...], w_ref[...],
                         preferred_element_type=jnp.float32)


def _matmul(x, w):
    m, k = x.shape
    _, n = w.shape
    bm = 512
    return pl.pallas_call(
        _mm_kernel,
        grid=(m // bm,),
        in_specs=[
            pl.BlockSpec((bm, k), lambda i: (i, 0)),
            pl.BlockSpec((k, n), lambda i: (0, 0)),
        ],
        out_specs=pl.BlockSpec((bm, n), lambda i: (i, 0)),
        out_shape=jax.ShapeDtypeStruct((m, n), jnp.float32),
    )(x, w)


def _mm_acc_kernel(a_ref, b_ref, o_ref):
    @pl.when(pl.program_id(0) == 0)
    def _():
        o_ref[...] = jnp.zeros_like(o_ref)

    o_ref[...] += jnp.dot(a_ref[...], b_ref[...],
                          preferred_element_type=jnp.float32)


def _matmul_kacc(a, b, bk):
    m, k = a.shape
    _, n = b.shape
    return pl.pallas_call(
        _mm_acc_kernel,
        grid=(k // bk,),
        in_specs=[
            pl.BlockSpec((m, bk), lambda i: (0, i)),
            pl.BlockSpec((bk, n), lambda i: (i, 0)),
        ],
        out_specs=pl.BlockSpec((m, n), lambda i: (0, 0)),
        out_shape=jax.ShapeDtypeStruct((m, n), jnp.float32),
    )(a, b)


def _head_kernel(pc_ref, w1_ref, b1_ref, w2_ref, b2_ref, o_ref):
    pc = pc_ref[...]
    cnt = jnp.clip(pc[:, HID:HID + 1], 1.0, None)
    pooled = pc[:, :HID] / cnt
    z = jnp.maximum(pooled @ w1_ref[...] + b1_ref[...], 0.0)
    o_ref[...] = (z @ w2_ref[...] + b2_ref[...]).T


def _mlp_head(pooled_cat, lin1_w, lin1_b, lin2_w, lin2_b):
    return pl.pallas_call(
        _head_kernel,
        out_shape=jax.ShapeDtypeStruct((1, G), jnp.float32),
    )(pooled_cat, lin1_w, lin1_b.reshape(1, -1), lin2_w,
      lin2_b.reshape(1, 1))


def _attn_body(src_ref, dst_ref, as_ref, ad_ref, ex_ref, denp_ref,
               sidx, didx, gs, gd, exv, zbuf, den_sh, sem0, sem1):
    cid = lax.axis_index("c")
    sid = lax.axis_index("s")
    wid = sid * NC + cid

    def zrow(i, _):
        zbuf[i] = jnp.zeros((16,), jnp.float32)
        return 0

    lax.fori_loop(0, 128, zrow, 0)
    for k in range(5):
        pltpu.sync_copy(zbuf, den_sh.at[pl.ds((sid * 5 + k) * 128, 128)])
    plsc.subcore_barrier()

    def tbody(t, _):
        chunk = t * NW + wid

        @pl.when(chunk < NCHUNK)
        def _():
            off = chunk * CH
            pltpu.sync_copy(src_ref.at[pl.ds(off, CH)], sidx)
            pltpu.sync_copy(dst_ref.at[pl.ds(off, CH)], didx)
            c1 = pltpu.async_copy(as_ref.at[sidx], gs, sem0)
            c2 = pltpu.async_copy(ad_ref.at[didx], gd, sem1)
            c1.wait()
            c2.wait()

            def ebody(e, _):
                v = gs[e] + gd[e]
                v = jnp.where(v >= 0.0, v, 0.2 * v)
                exv[e] = jnp.exp(v)
                return 0

            lax.fori_loop(0, CH, ebody, 0)
            pltpu.sync_copy(exv, ex_ref.at[pl.ds(off, CH)])
            pltpu.sync_copy(exv, den_sh.at[didx], add=True)

        return 0

    lax.fori_loop(0, (NCHUNK + NW - 1) // NW, tbody, 0)
    plsc.subcore_barrier()
    pltpu.sync_copy(den_sh.at[pl.ds(sid * ROWS_W, ROWS_W)],
                    denp_ref.at[pl.ds(cid * NP + sid * ROWS_W, ROWS_W)])


@functools.partial(
    pl.kernel,
    out_type=(jax.ShapeDtypeStruct((E, 16), jnp.float32),
              jax.ShapeDtypeStruct((NC * NP, 16), jnp.float32)),
    mesh=_mesh,
    compiler_params=pltpu.CompilerParams(use_tc_tiling_on_sc=False),
    scratch_types=[
        pltpu.VMEM((CH,), jnp.int32),
        pltpu.VMEM((CH,), jnp.int32),
        pltpu.VMEM((CH, 16), jnp.float32),
        pltpu.VMEM((CH, 16), jnp.float32),
        pltpu.VMEM((CH, 16), jnp.float32),
        pltpu.VMEM((128, 16), jnp.float32),
        pltpu.VMEM_SHARED((NP, 16), jnp.float32),
        pltpu.SemaphoreType.DMA,
        pltpu.SemaphoreType.DMA,
    ],
)
def _sc_attention(src_ref, dst_ref, as_ref, ad_ref, ex_ref, denp_ref,
                  sidx, didx, gs, gd, exv, zbuf, den_sh, sem0, sem1):
    _attn_body(src_ref, dst_ref, as_ref, ad_ref, ex_ref, denp_ref,
               sidx, didx, gs, gd, exv, zbuf, den_sh, sem0, sem1)


def _make_aggregate(D, nheads, col0):
    vh = D // nheads // 16

    def body(src_ref, dst_ref, ex_ref, h_ref, up_ref,
             sidx, didx, exch, rows, zbuf, u_sh, sem0):
        cid = lax.axis_index("c")
        sid = lax.axis_index("s")
        wid = sid * NC + cid

        def zrow(i, _):
            for j in range(D // 16):
                zbuf[i, pl.ds(j * 16, 16)] = jnp.zeros((16,), jnp.float32)
            return 0

        lax.fori_loop(0, 128, zrow, 0)
        for k in range(5):
            pltpu.sync_copy(zbuf, u_sh.at[pl.ds((sid * 5 + k) * 128, 128)])
        plsc.subcore_barrier()

        def tbody(t, _):
            chunk = t * NW + wid

            @pl.when(chunk < NCHUNK)
            def _():
                off = chunk * CH
                pltpu.sync_copy(src_ref.at[pl.ds(off, CH)], sidx)
                pltpu.sync_copy(dst_ref.at[pl.ds(off, CH)], didx)
                pltpu.sync_copy(ex_ref.at[pl.ds(off, CH)], exch)
                pltpu.async_copy(h_ref.at[sidx], rows, sem0).wait()

                def ebody(e, _):
                    exrow = exch[e]
                    for hh in range(nheads):
                        s = exrow[col0 + hh]
                        for j in range(vh):
                            sl = pl.ds((hh * vh + j) * 16, 16)
                            rows[e, sl] = rows[e, sl] * s
                    return 0

                lax.fori_loop(0, CH, ebody, 0)
                pltpu.sync_copy(rows, u_sh.at[didx], add=True)

            return 0

        lax.fori_loop(0, (NCHUNK + NW - 1) // NW, tbody, 0)
        plsc.subcore_barrier()
        pltpu.sync_copy(u_sh.at[pl.ds(sid * ROWS_W, ROWS_W)],
                        up_ref.at[pl.ds(cid * NP + sid * ROWS_W, ROWS_W)])

    return pl.kernel(
        body,
        out_type=jax.ShapeDtypeStruct((NC * NP, D), jnp.float32),
        mesh=_mesh,
        compiler_params=pltpu.CompilerParams(use_tc_tiling_on_sc=False),
        scratch_types=[
            pltpu.VMEM((CH,), jnp.int32),
            pltpu.VMEM((CH,), jnp.int32),
            pltpu.VMEM((CH, 16), jnp.float32),
            pltpu.VMEM((CH, D), jnp.float32),
            pltpu.VMEM((128, D), jnp.float32),
            pltpu.VMEM_SHARED((NP, D), jnp.float32),
            pltpu.SemaphoreType.DMA,
        ],
    )


_agg_l1 = [_make_aggregate(128, 2, 2 * c) for c in range(4)]
_agg_l2 = _make_aggregate(64, 1, 0)


def _pad16(a):
    return jnp.pad(a, ((0, 0), (0, 16 - a.shape[1])))


def _gat_layer(src, dst, h, a_src, a_dst, b, heads, agg_kernels):
    ex, denp = _sc_attention(src, dst, _pad16(a_src), _pad16(a_dst))
    den = denp[:N, :heads] + denp[NP:NP + N, :heads]

    chunk_w = h.shape[1] // len(agg_kernels)
    u_parts = []
    for c, agg in enumerate(agg_kernels):
        hp = agg(src, dst, ex, h[:, c * chunk_w:(c + 1) * chunk_w])
        u_parts.append(hp[:N] + hp[NP:NP + N])
    u = jnp.concatenate(u_parts, axis=1)

    exself = jnp.exp(jax.nn.leaky_relu(a_src + a_dst, negative_slope=0.2))
    u = u + (exself[:, :, None] * h.reshape(N, heads, HID)).reshape(N, -1)
    den = den + exself
    out = u.reshape(N, heads, HID) / (den[:, :, None] + 1e-16)
    return out.reshape(N, heads * HID) + b[None, :]


def kernel(x, edge_index, edge_attr, batch, W1, att_src1, att_dst1, b1,
           W2, att_src2, att_dst2, b2, lin1_w, lin1_b, lin2_w, lin2_b):
    src = edge_index[0].astype(jnp.int32)
    dst = edge_index[1].astype(jnp.int32)

    npad = 10240
    xp = jnp.pad(x, ((0, npad - N), (0, 0)))
    hh = jnp.arange(HEADS * HID) // HID
    blk_s = jnp.where(hh[:, None] == jnp.arange(HEADS)[None, :],
                      att_src1.reshape(-1)[:, None], 0.0)
    blk_d = jnp.where(hh[:, None] == jnp.arange(HEADS)[None, :],
                      att_dst1.reshape(-1)[:, None], 0.0)
    wext = jnp.concatenate(
        [W1, W1 @ blk_s, W1 @ blk_d,
         jnp.zeros((IN, 112), jnp.float32)], axis=1)
    he = _matmul(xp, wext)
    h1 = he[:N, :HEADS * HID]
    as1 = he[:N, 512:520]
    ad1 = he[:N, 520:528]

    g1 = jax.nn.elu(_gat_layer(src, dst, h1, as1, ad1, b1, HEADS, _agg_l1))

    g1p = jnp.pad(g1, ((0, npad - N), (0, 0)))
    wext2 = jnp.concatenate(
        [W2, (W2 @ att_src2[0])[:, None], (W2 @ att_dst2[0])[:, None],
         jnp.zeros((HEADS * HID, 62), jnp.float32)], axis=1)
    he2 = _matmul(g1p, wext2)
    h2 = he2[:N, :HID]
    as2 = he2[:N, HID:HID + 1]
    ad2 = he2[:N, HID + 1:HID + 2]

    g2 = jax.nn.elu(_gat_layer(src, dst, h2, as2, ad2, b2, 1, [_agg_l2]))

    bpad = jnp.pad(batch, (0, npad - N), constant_values=2 * G)
    onehot = (bpad[None, :] == jnp.arange(G)[:, None]).astype(jnp.float32)
    x2 = jnp.concatenate(
        [g2, jnp.ones((N, 1), jnp.float32),
         jnp.zeros((N, 128 - HID - 1), jnp.float32)], axis=1)
    x2 = jnp.pad(x2, ((0, npad - N), (0, 0)))
    pooled_cat = _matmul_kacc(onehot, x2, 1024)
    out = _mlp_head(pooled_cat, lin1_w, lin1_b, lin2_w, lin2_b)
    return out.reshape(G)
